# Initial kernel scaffold; baseline (speedup 1.0000x reference)
import functools

import jax
import jax.numpy as jnp
from jax import lax
from jax.experimental import pallas as pl
from jax.experimental.pallas import tpu as pltpu
from jax.experimental.pallas import tpu_sc as plsc

BATCH = 16384
VOCAB = 1000
EMB_DIM = 64
EPS = 1e-5

_info = plsc.get_sparse_core_info()
_NC, _NS, _NL = _info.num_cores, _info.num_subcores, _info.num_lanes
_NW = _NC * _NS
_BPW = BATCH // _NW


def _prep_body(ages_ref, emb_ref, w1_ref, wage_ref, b_ref, g_ref, bt_ref,
               p_ref, u_ref):
    a = ages_ref[...]
    mean = jnp.mean(a)
    var = jnp.mean((a - mean) ** 2)
    scale = g_ref[0, 0] * lax.rsqrt(var + EPS)
    c0 = bt_ref[0, 0] - mean * scale
    wage = wage_ref[...]
    p = lax.dot_general(emb_ref[...], w1_ref[...],
                        (((1,), (1,)), ((), ())),
                        preferred_element_type=jnp.float32)
    p_ref[...] = p + b_ref[...] + c0 * wage
    u_ref[...] = scale * wage


_prep = pl.pallas_call(
    _prep_body,
    out_shape=[
        jax.ShapeDtypeStruct((VOCAB, EMB_DIM), jnp.float32),
        jax.ShapeDtypeStruct((1, EMB_DIM), jnp.float32),
    ],
)


def _sc_body(ids_hbm, ages_hbm, u_hbm, p_hbm, out_hbm,
             idx_v, ages_v, u_v, rows_v, sem):
    wid = lax.axis_index("s") * _NC + lax.axis_index("c")
    base = wid * _BPW
    pltpu.sync_copy(ids_hbm.at[pl.ds(base, _BPW)], idx_v)
    pltpu.sync_copy(ages_hbm.at[pl.ds(base, _BPW)], ages_v)
    pltpu.sync_copy(u_hbm, u_v)
    gat = pltpu.async_copy(p_hbm.at[idx_v], rows_v, sem)
    u0 = u_v[pl.ds(0, 16)]
    u1 = u_v[pl.ds(16, 16)]
    u2 = u_v[pl.ds(32, 16)]
    u3 = u_v[pl.ds(48, 16)]
    gat.wait()

    def body(j, carry):
        a = ages_v[j]
        rows_v[j, pl.ds(0, 16)] = rows_v[j, pl.ds(0, 16)] + a * u0
        rows_v[j, pl.ds(16, 16)] = rows_v[j, pl.ds(16, 16)] + a * u1
        rows_v[j, pl.ds(32, 16)] = rows_v[j, pl.ds(32, 16)] + a * u2
        rows_v[j, pl.ds(48, 16)] = rows_v[j, pl.ds(48, 16)] + a * u3
        return carry

    lax.fori_loop(0, _BPW, body, 0, unroll=4)
    pltpu.sync_copy(rows_v, out_hbm.at[pl.ds(base, _BPW)])


_sc_gather = functools.partial(
    pl.kernel,
    mesh=plsc.VectorSubcoreMesh(core_axis_name="c", subcore_axis_name="s"),
    out_type=jax.ShapeDtypeStruct((BATCH, EMB_DIM), jnp.float32),
    scratch_types=[
        pltpu.VMEM((_BPW,), jnp.int32),
        pltpu.VMEM((_BPW,), jnp.float32),
        pltpu.VMEM((EMB_DIM,), jnp.float32),
        pltpu.VMEM((_BPW, EMB_DIM), jnp.float32),
        pltpu.SemaphoreType.DMA,
    ],
)(_sc_body)


def kernel(customer_ids, ages, emb_table, bn_gamma, bn_beta, W, b):
    ids = customer_ids.astype(jnp.int32)
    w1 = W[:, :EMB_DIM]
    wage = W[:, EMB_DIM].reshape(1, EMB_DIM)
    ages2 = ages.reshape(128, 128)
    p, u = _prep(ages2, emb_table, w1, wage, b.reshape(1, EMB_DIM),
                 bn_gamma.reshape(1, 1), bn_beta.reshape(1, 1))
    return _sc_gather(ids, ages, u.reshape(EMB_DIM), p)


# speedup vs baseline: 1.4931x; 1.4931x over previous
import functools

import jax
import jax.numpy as jnp
from jax import lax
from jax.experimental import pallas as pl
from jax.experimental.pallas import tpu as pltpu
from jax.experimental.pallas import tpu_sc as plsc

BATCH = 16384
VOCAB = 1000
EMB_DIM = 64
EPS = 1e-5

_info = plsc.get_sparse_core_info()
_NC, _NS, _NL = _info.num_cores, _info.num_subcores, _info.num_lanes
_NW = _NC * _NS
_BPW = BATCH // _NW


def _prep_body(ages_ref, emb_ref, w1_ref, wage_ref, b_ref, g_ref, bt_ref,
               p_ref, u_ref):
    a = ages_ref[...]
    mean = jnp.mean(a)
    var = jnp.mean((a - mean) ** 2)
    scale = g_ref[0, 0] * lax.rsqrt(var + EPS)
    c0 = bt_ref[0, 0] - mean * scale
    wage = wage_ref[...]
    p = lax.dot_general(emb_ref[...], w1_ref[...],
                        (((1,), (1,)), ((), ())),
                        preferred_element_type=jnp.float32)
    p_ref[...] = p + b_ref[...] + c0 * wage
    u_ref[...] = scale * wage


_prep = pl.pallas_call(
    _prep_body,
    out_shape=[
        jax.ShapeDtypeStruct((VOCAB, EMB_DIM), jnp.float32),
        jax.ShapeDtypeStruct((1, EMB_DIM), jnp.float32),
    ],
)


def _sc_body(ids_hbm, ages_hbm, u_hbm, p_hbm, out_hbm,
             idx_v, ages_v, u_v, rows_v, sem):
    wid = lax.axis_index("s") * _NC + lax.axis_index("c")
    base = wid * _BPW
    pltpu.sync_copy(ids_hbm.at[pl.ds(base, _BPW)], idx_v)
    pltpu.sync_copy(ages_hbm.at[pl.ds(base, _BPW)], ages_v)
    pltpu.sync_copy(u_hbm, u_v)
    gat = pltpu.async_copy(p_hbm.at[idx_v], rows_v, sem)
    u0 = u_v[pl.ds(0, 16)]
    u1 = u_v[pl.ds(16, 16)]
    u2 = u_v[pl.ds(32, 16)]
    u3 = u_v[pl.ds(48, 16)]
    gat.wait()

    def body(t, carry):
        base16 = t * _NL
        a16 = ages_v[pl.ds(base16, _NL)]
        for jj in range(_NL):
            j = base16 + jj
            a = a16[jj]
            rows_v[j, pl.ds(0, 16)] = rows_v[j, pl.ds(0, 16)] + a * u0
            rows_v[j, pl.ds(16, 16)] = rows_v[j, pl.ds(16, 16)] + a * u1
            rows_v[j, pl.ds(32, 16)] = rows_v[j, pl.ds(32, 16)] + a * u2
            rows_v[j, pl.ds(48, 16)] = rows_v[j, pl.ds(48, 16)] + a * u3
        return carry

    lax.fori_loop(0, _BPW // _NL, body, 0)
    pltpu.sync_copy(rows_v, out_hbm.at[pl.ds(base, _BPW)])


_sc_gather = functools.partial(
    pl.kernel,
    mesh=plsc.VectorSubcoreMesh(core_axis_name="c", subcore_axis_name="s"),
    out_type=jax.ShapeDtypeStruct((BATCH, EMB_DIM), jnp.float32),
    scratch_types=[
        pltpu.VMEM((_BPW,), jnp.int32),
        pltpu.VMEM((_BPW,), jnp.float32),
        pltpu.VMEM((EMB_DIM,), jnp.float32),
        pltpu.VMEM((_BPW, EMB_DIM), jnp.float32),
        pltpu.SemaphoreType.DMA,
    ],
    compiler_params=pltpu.CompilerParams(use_tc_tiling_on_sc=False),
)(_sc_body)


def kernel(customer_ids, ages, emb_table, bn_gamma, bn_beta, W, b):
    ids = customer_ids.astype(jnp.int32)
    w1 = W[:, :EMB_DIM]
    wage = W[:, EMB_DIM].reshape(1, EMB_DIM)
    ages2 = ages.reshape(128, 128)
    p, u = _prep(ages2, emb_table, w1, wage, b.reshape(1, EMB_DIM),
                 bn_gamma.reshape(1, 1), bn_beta.reshape(1, 1))
    return _sc_gather(ids, ages, u.reshape(EMB_DIM), p)
